# Initial kernel scaffold; baseline (speedup 1.0000x reference)
import jax
import jax.numpy as jnp
from jax.experimental import pallas as pl


def kernel(h, tau):
    raise NotImplementedError("write your pallas kernel here")


# speedup vs baseline: 41.0966x; 41.0966x over previous
import jax
import jax.numpy as jnp
from jax.experimental import pallas as pl
from jax.experimental.pallas import tpu as pltpu

_SPARSITY = 0.1
_LANES = 1024
_SEARCH_BITS = 31


def _body(k, tau_ref, h_hbm, o_hbm, buf, ubits, sem_in, sem_out):
    n = pl.program_id(0)
    cp_in = pltpu.make_async_copy(h_hbm.at[n], buf, sem_in)
    cp_in.start()
    cp_in.wait()

    ubits[...] = jax.lax.bitcast_convert_type(jnp.abs(buf[...]), jnp.int32)

    def step(_, carry):
        lo, hi = carry
        mid = lo + ((hi - lo) >> 1)
        cnt = jnp.sum(jnp.where(ubits[...] >= mid, 1.0, 0.0))
        big = cnt >= k
        return jnp.where(big, mid, lo), jnp.where(big, hi, mid)

    lo, _ = jax.lax.fori_loop(
        0, _SEARCH_BITS, step,
        (jnp.int32(0), jnp.int32(0x7F800000)),
    )
    m = jnp.where(ubits[...] >= lo, 1.0, 0.0)
    tau = tau_ref[0, 0]
    buf[...] = buf[...] * (m * tau + (1.0 - tau))

    cp_out = pltpu.make_async_copy(buf, o_hbm.at[n], sem_out)
    cp_out.start()
    cp_out.wait()


def kernel(h, tau):
    N, C, H, W = h.shape
    total = C * H * W
    idx = int(_SPARSITY * C * H * W)
    k = idx + 1
    assert total % _LANES == 0
    R = total // _LANES
    hr = h.reshape(N, R, _LANES)
    tau_arr = jnp.asarray(tau, jnp.float32).reshape(1, 1)

    out = pl.pallas_call(
        lambda tau_ref, h_ref, o_ref, buf, ubits, s1, s2: _body(
            k, tau_ref, h_ref, o_ref, buf, ubits, s1, s2),
        grid=(N,),
        in_specs=[
            pl.BlockSpec((1, 1), lambda n: (0, 0)),
            pl.BlockSpec(memory_space=pl.ANY),
        ],
        out_specs=pl.BlockSpec(memory_space=pl.ANY),
        out_shape=jax.ShapeDtypeStruct((N, R, _LANES), jnp.float32),
        scratch_shapes=[
            pltpu.VMEM((R, _LANES), jnp.float32),
            pltpu.VMEM((R, _LANES), jnp.int32),
            pltpu.SemaphoreType.DMA,
            pltpu.SemaphoreType.DMA,
        ],
    )(tau_arr, hr)
    return out.reshape(N, C, H, W)


# speedup vs baseline: 50.4556x; 1.2277x over previous
import jax
import jax.numpy as jnp
from jax.experimental import pallas as pl
from jax.experimental.pallas import tpu as pltpu

_SPARSITY = 0.1
_LANES = 1024
_BR = 16
_SEARCH_BITS = 31


def _body(k, R, tau_ref, h_hbm, o_hbm, buf, ubits, sem_in, sem_out):
    n = pl.program_id(0)
    nch = R // _BR
    cp_in = pltpu.make_async_copy(h_hbm.at[n], buf, sem_in)
    cp_in.start()
    cp_in.wait()

    def init(i, c):
        ubits[pl.ds(i * _BR, _BR), :] = jax.lax.bitcast_convert_type(
            jnp.abs(buf[pl.ds(i * _BR, _BR), :]), jnp.int32)
        return c
    jax.lax.fori_loop(0, nch, init, 0)

    def count_ge(mid):
        def acc_body(i, acc):
            blk = ubits[pl.ds(i * _BR, _BR), :]
            return acc + jnp.where(blk >= mid, 1.0, 0.0)
        acc = jax.lax.fori_loop(
            0, nch, acc_body, jnp.zeros((_BR, _LANES), jnp.float32))
        return jnp.sum(acc)

    def step(_, carry):
        lo, hi = carry
        mid = lo + ((hi - lo) >> 1)
        big = count_ge(mid) >= k
        return jnp.where(big, mid, lo), jnp.where(big, hi, mid)

    lo, _ = jax.lax.fori_loop(
        0, _SEARCH_BITS, step,
        (jnp.int32(0), jnp.int32(0x7F800000)),
    )

    tau = tau_ref[0, 0]
    def fin(i, c):
        sl = pl.ds(i * _BR, _BR)
        m = jnp.where(ubits[sl, :] >= lo, 1.0, 0.0)
        buf[sl, :] = buf[sl, :] * (m * tau + (1.0 - tau))
        return c
    jax.lax.fori_loop(0, nch, fin, 0)

    cp_out = pltpu.make_async_copy(buf, o_hbm.at[n], sem_out)
    cp_out.start()
    cp_out.wait()


def kernel(h, tau):
    N, C, H, W = h.shape
    total = C * H * W
    idx = int(_SPARSITY * C * H * W)
    k = idx + 1
    assert total % (_LANES * _BR) == 0
    R = total // _LANES
    hr = h.reshape(N, R, _LANES)
    tau_arr = jnp.asarray(tau, jnp.float32).reshape(1, 1)

    out = pl.pallas_call(
        lambda tau_ref, h_ref, o_ref, buf, ubits, s1, s2: _body(
            k, R, tau_ref, h_ref, o_ref, buf, ubits, s1, s2),
        grid=(N,),
        in_specs=[
            pl.BlockSpec((1, 1), lambda n: (0, 0)),
            pl.BlockSpec(memory_space=pl.ANY),
        ],
        out_specs=pl.BlockSpec(memory_space=pl.ANY),
        out_shape=jax.ShapeDtypeStruct((N, R, _LANES), jnp.float32),
        scratch_shapes=[
            pltpu.VMEM((R, _LANES), jnp.float32),
            pltpu.VMEM((R, _LANES), jnp.int32),
            pltpu.SemaphoreType.DMA,
            pltpu.SemaphoreType.DMA,
        ],
    )(tau_arr, hr)
    return out.reshape(N, C, H, W)


# speedup vs baseline: 67.7017x; 1.3418x over previous
import jax
import jax.numpy as jnp
from jax.experimental import pallas as pl
from jax.experimental.pallas import tpu as pltpu

_SPARSITY = 0.1
_LANES = 1024
_BR = 16
_SEARCH_BITS = 31


def _body(k, R, tau_ref, h_hbm, o_hbm, buf, ubits, sem_in, sem_out):
    n = pl.program_id(0)
    nch = R // _BR
    cp_in = pltpu.make_async_copy(h_hbm.at[n], buf, sem_in)
    cp_in.start()
    cp_in.wait()

    def init(i, c):
        ubits[pl.ds(i * _BR, _BR), :] = jax.lax.bitcast_convert_type(
            jnp.abs(buf[pl.ds(i * _BR, _BR), :]), jnp.int32)
        return c
    jax.lax.fori_loop(0, nch, init, 0, unroll=7)

    def count_ge(mid):
        def acc_body(i, acc):
            blk = ubits[pl.ds(i * _BR, _BR), :]
            return acc + jnp.where(blk >= mid, 1.0, 0.0)
        acc = jax.lax.fori_loop(
            0, nch, acc_body, jnp.zeros((_BR, _LANES), jnp.float32),
            unroll=7)
        return jnp.sum(acc)

    def step(_, carry):
        lo, hi = carry
        mid = lo + ((hi - lo) >> 1)
        big = count_ge(mid) >= k
        return jnp.where(big, mid, lo), jnp.where(big, hi, mid)

    lo, _ = jax.lax.fori_loop(
        0, _SEARCH_BITS, step,
        (jnp.int32(0), jnp.int32(0x7F800000)),
    )

    tau = tau_ref[0, 0]
    def fin(i, c):
        sl = pl.ds(i * _BR, _BR)
        m = jnp.where(ubits[sl, :] >= lo, 1.0, 0.0)
        buf[sl, :] = buf[sl, :] * (m * tau + (1.0 - tau))
        return c
    jax.lax.fori_loop(0, nch, fin, 0, unroll=7)

    cp_out = pltpu.make_async_copy(buf, o_hbm.at[n], sem_out)
    cp_out.start()
    cp_out.wait()


def kernel(h, tau):
    N, C, H, W = h.shape
    total = C * H * W
    idx = int(_SPARSITY * C * H * W)
    k = idx + 1
    assert total % (_LANES * _BR) == 0
    R = total // _LANES
    hr = h.reshape(N, R, _LANES)
    tau_arr = jnp.asarray(tau, jnp.float32).reshape(1, 1)

    out = pl.pallas_call(
        lambda tau_ref, h_ref, o_ref, buf, ubits, s1, s2: _body(
            k, R, tau_ref, h_ref, o_ref, buf, ubits, s1, s2),
        grid=(N,),
        in_specs=[
            pl.BlockSpec((1, 1), lambda n: (0, 0)),
            pl.BlockSpec(memory_space=pl.ANY),
        ],
        out_specs=pl.BlockSpec(memory_space=pl.ANY),
        out_shape=jax.ShapeDtypeStruct((N, R, _LANES), jnp.float32),
        scratch_shapes=[
            pltpu.VMEM((R, _LANES), jnp.float32),
            pltpu.VMEM((R, _LANES), jnp.int32),
            pltpu.SemaphoreType.DMA,
            pltpu.SemaphoreType.DMA,
        ],
    )(tau_arr, hr)
    return out.reshape(N, C, H, W)


# speedup vs baseline: 74.9559x; 1.1071x over previous
import jax
import jax.numpy as jnp
from jax.experimental import pallas as pl
from jax.experimental.pallas import tpu as pltpu

_SPARSITY = 0.1
_LANES = 1024
_BR = 16
_SUB_ROWS = 256
_SEARCH_BITS = 31


def _body(k, R, tau_ref, h_hbm, o_hbm, buf, ubits, sem_in, sem_out):
    n = pl.program_id(0)
    nch = R // _BR
    sub_rows = min(_SUB_ROWS, R)
    cp_in = pltpu.make_async_copy(h_hbm.at[n], buf, sem_in)
    cp_in.start()
    cp_in.wait()

    def init(i, c):
        ubits[pl.ds(i * _BR, _BR), :] = jax.lax.bitcast_convert_type(
            jnp.abs(buf[pl.ds(i * _BR, _BR), :]), jnp.int32)
        return c
    jax.lax.fori_loop(0, nch, init, 0, unroll=7)

    def count_ge(mid):
        def acc_body(i, acc):
            blk = ubits[pl.ds(i * _BR, _BR), :]
            return acc + jnp.where(blk >= mid, 1.0, 0.0)
        acc = jax.lax.fori_loop(
            0, nch, acc_body, jnp.zeros((_BR, _LANES), jnp.float32),
            unroll=7)
        return jnp.sum(acc)

    sub_n = sub_rows * _LANES
    ks = k * sub_n // (R * _LANES)
    slack = 540
    kA = jnp.float32(ks + slack)
    kB = jnp.float32(max(ks - slack, 0))

    def sub_count2(mA, mB):
        def acc_body(i, accs):
            aA, aB = accs
            blk = ubits[pl.ds(i * _BR, _BR), :]
            aA = aA + jnp.where(blk >= mA, 1.0, 0.0)
            aB = aB + jnp.where(blk >= mB, 1.0, 0.0)
            return aA, aB
        z = jnp.zeros((_BR, _LANES), jnp.float32)
        aA, aB = jax.lax.fori_loop(
            0, sub_rows // _BR, acc_body, (z, z), unroll=4)
        return jnp.sum(aA), jnp.sum(aB)

    def sub_step(_, carry):
        loA, hiA, loB, hiB = carry
        mA = loA + ((hiA - loA) >> 1)
        mB = loB + ((hiB - loB) >> 1)
        cA, cB = sub_count2(mA, mB)
        bA = cA >= kA
        bB = cB >= kB
        return (jnp.where(bA, mA, loA), jnp.where(bA, hiA, mA),
                jnp.where(bB, mB, loB), jnp.where(bB, hiB, mB))

    z0 = jnp.int32(0)
    z1 = jnp.int32(0x7F800000)
    loA, _, loB, _ = jax.lax.fori_loop(
        0, _SEARCH_BITS, sub_step, (z0, z1, z0, z1))
    hi_cand = loB + 1

    c_lo = count_ge(loA)
    c_hi = count_ge(hi_cand)
    lo0 = jnp.where(c_lo >= k, loA, z0)
    hi0 = jnp.where(c_hi < k, hi_cand, z1)

    def w_cond(carry):
        lo, hi = carry
        return hi - lo > 1

    def w_body(carry):
        lo, hi = carry
        mid = lo + ((hi - lo) >> 1)
        big = count_ge(mid) >= k
        return jnp.where(big, mid, lo), jnp.where(big, hi, mid)

    lo, _ = jax.lax.while_loop(w_cond, w_body, (lo0, hi0))

    tau = tau_ref[0, 0]
    def fin(i, c):
        sl = pl.ds(i * _BR, _BR)
        m = jnp.where(ubits[sl, :] >= lo, 1.0, 0.0)
        buf[sl, :] = buf[sl, :] * (m * tau + (1.0 - tau))
        return c
    jax.lax.fori_loop(0, nch, fin, 0, unroll=7)

    cp_out = pltpu.make_async_copy(buf, o_hbm.at[n], sem_out)
    cp_out.start()
    cp_out.wait()


def kernel(h, tau):
    N, C, H, W = h.shape
    total = C * H * W
    idx = int(_SPARSITY * C * H * W)
    k = idx + 1
    assert total % (_LANES * _BR) == 0
    R = total // _LANES
    hr = h.reshape(N, R, _LANES)
    tau_arr = jnp.asarray(tau, jnp.float32).reshape(1, 1)

    out = pl.pallas_call(
        lambda tau_ref, h_ref, o_ref, buf, ubits, s1, s2: _body(
            k, R, tau_ref, h_ref, o_ref, buf, ubits, s1, s2),
        grid=(N,),
        in_specs=[
            pl.BlockSpec((1, 1), lambda n: (0, 0)),
            pl.BlockSpec(memory_space=pl.ANY),
        ],
        out_specs=pl.BlockSpec(memory_space=pl.ANY),
        out_shape=jax.ShapeDtypeStruct((N, R, _LANES), jnp.float32),
        scratch_shapes=[
            pltpu.VMEM((R, _LANES), jnp.float32),
            pltpu.VMEM((R, _LANES), jnp.int32),
            pltpu.SemaphoreType.DMA,
            pltpu.SemaphoreType.DMA,
        ],
    )(tau_arr, hr)
    return out.reshape(N, C, H, W)


# speedup vs baseline: 85.1577x; 1.1361x over previous
import jax
import jax.numpy as jnp
from jax.experimental import pallas as pl
from jax.experimental.pallas import tpu as pltpu

_SPARSITY = 0.1
_LANES = 1024
_BR = 16
_SUB_ROWS = 128
_SEARCH_BITS = 31
_REFINE = 17


def _bits_f(v):
    return jax.lax.bitcast_convert_type(v, jnp.float32)


def _run(k, R, N, tau_ref, h_hbm, o_hbm, cur, bits, pre,
         sin_cur, sin_pre, sout_self, sout_prev):
    n = pl.program_id(0)
    nch = R // _BR
    sub_rows = min(_SUB_ROWS, R)

    @pl.when(n == 0)
    def _():
        pltpu.make_async_copy(h_hbm.at[n], cur, sin_cur).start()

    pltpu.make_async_copy(h_hbm.at[n], cur, sin_cur).wait()

    def init(i, c):
        sl = pl.ds(i * _BR, _BR)
        bits[sl, :] = jnp.abs(cur[sl, :])
        return c
    jax.lax.fori_loop(0, nch, init, 0, unroll=7)

    kkv = jnp.full((1, 1), float(k), jnp.float32)
    z0 = jnp.full((1, 1), 0, jnp.int32)
    z1 = jnp.full((1, 1), 0x7F800000, jnp.int32)

    def count1(rows, mid_f):
        def acc_body(i, acc):
            blk = bits[pl.ds(i * _BR, _BR), :]
            return acc + jnp.where(blk >= mid_f, 1.0, 0.0)
        acc = jax.lax.fori_loop(
            0, rows // _BR, acc_body,
            jnp.zeros((_BR, _LANES), jnp.float32), unroll=7)
        return jnp.sum(acc, axis=(0, 1), keepdims=True)

    def count2(rows, mA_f, mB_f, unroll):
        def acc_body(i, accs):
            aA, aB = accs
            blk = bits[pl.ds(i * _BR, _BR), :]
            aA = aA + jnp.where(blk >= mA_f, 1.0, 0.0)
            aB = aB + jnp.where(blk >= mB_f, 1.0, 0.0)
            return aA, aB
        z = jnp.zeros((_BR, _LANES), jnp.float32)
        aA, aB = jax.lax.fori_loop(0, rows // _BR, acc_body, (z, z),
                                   unroll=unroll)
        return (jnp.sum(aA, axis=(0, 1), keepdims=True),
                jnp.sum(aB, axis=(0, 1), keepdims=True))

    ks = k * (sub_rows * _LANES) // (R * _LANES)
    slack = 380
    kAv = jnp.full((1, 1), float(ks + slack), jnp.float32)
    kBv = jnp.full((1, 1), float(max(ks - slack, 0)), jnp.float32)

    def sub_step(_, carry):
        loA, hiA, loB, hiB = carry
        mA = loA + ((hiA - loA) >> 1)
        mB = loB + ((hiB - loB) >> 1)
        cA, cB = count2(sub_rows, _bits_f(mA), _bits_f(mB), 4)
        bA = cA >= kAv
        bB = cB >= kBv
        return (jnp.where(bA, mA, loA), jnp.where(bA, hiA, mA),
                jnp.where(bB, mB, loB), jnp.where(bB, hiB, mB))

    loA, _, loB, _ = jax.lax.fori_loop(
        0, _SEARCH_BITS, sub_step, (z0, z1, z0, z1))
    hi_cand = loB + 1

    @pl.when(n >= 1)
    def _():
        pltpu.make_async_copy(pre, o_hbm.at[n - 1], sout_prev).wait()

    @pl.when(n + 1 < N)
    def _():
        pltpu.make_async_copy(h_hbm.at[n + 1], pre, sin_pre).start()

    c_lo, c_hi = count2(R, _bits_f(loA), _bits_f(hi_cand), 7)
    lo = jnp.where(c_lo >= kkv, loA, z0)
    hi = jnp.where(c_hi < kkv, hi_cand, z1)

    def step(carry):
        lo, hi = carry
        mid = lo + ((hi - lo) >> 1)
        big = count1(R, _bits_f(mid)) >= kkv
        return jnp.where(big, mid, lo), jnp.where(big, hi, mid)

    lo, hi = jax.lax.fori_loop(0, _REFINE, lambda i, c: step(c), (lo, hi))

    lo_s, hi_s = lo[0, 0], hi[0, 0]

    def w_cond(carry):
        lo, hi = carry
        return hi - lo > 1

    def w_body(carry):
        lo, hi = carry
        l2 = jnp.full((1, 1), 1, jnp.int32) * lo
        h2 = jnp.full((1, 1), 1, jnp.int32) * hi
        l2, h2 = step((l2, h2))
        return l2[0, 0], h2[0, 0]

    lo_s, _ = jax.lax.while_loop(w_cond, w_body, (lo_s, hi_s))

    cutoff_f = _bits_f(jnp.full((1, 1), 1, jnp.int32) * lo_s)
    tau = tau_ref[0, 0]
    one_minus_tau = 1.0 - tau

    def fin(i, c):
        sl = pl.ds(i * _BR, _BR)
        f = jnp.where(bits[sl, :] >= cutoff_f, 1.0, one_minus_tau)
        bits[sl, :] = cur[sl, :] * f
        return c
    jax.lax.fori_loop(0, nch, fin, 0, unroll=7)

    pltpu.make_async_copy(bits, o_hbm.at[n], sout_self).start()

    @pl.when(n == N - 1)
    def _():
        pltpu.make_async_copy(bits, o_hbm.at[n], sout_self).wait()


def _body(k, R, N, tau_ref, h_hbm, o_hbm, b0, b1, b2,
          si0, si1, si2, so0, so1, so2):
    n = pl.program_id(0)
    bufs = (b0, b1, b2)
    sins = (si0, si1, si2)
    souts = (so0, so1, so2)
    for r in range(3):
        @pl.when(n % 3 == r)
        def _(r=r):
            _run(k, R, N, tau_ref, h_hbm, o_hbm,
                 bufs[r], bufs[(r + 2) % 3], bufs[(r + 1) % 3],
                 sins[r], sins[(r + 1) % 3],
                 souts[(r + 2) % 3], souts[(r + 1) % 3])


def kernel(h, tau):
    N, C, H, W = h.shape
    total = C * H * W
    idx = int(_SPARSITY * C * H * W)
    k = idx + 1
    assert total % (_LANES * _BR) == 0
    R = total // _LANES
    hr = h.reshape(N, R, _LANES)
    tau_arr = jnp.asarray(tau, jnp.float32).reshape(1, 1)

    out = pl.pallas_call(
        lambda *refs: _body(k, R, N, *refs),
        grid=(N,),
        in_specs=[
            pl.BlockSpec((1, 1), lambda n: (0, 0)),
            pl.BlockSpec(memory_space=pl.ANY),
        ],
        out_specs=pl.BlockSpec(memory_space=pl.ANY),
        out_shape=jax.ShapeDtypeStruct((N, R, _LANES), jnp.float32),
        scratch_shapes=[
            pltpu.VMEM((R, _LANES), jnp.float32),
            pltpu.VMEM((R, _LANES), jnp.float32),
            pltpu.VMEM((R, _LANES), jnp.float32),
            pltpu.SemaphoreType.DMA,
            pltpu.SemaphoreType.DMA,
            pltpu.SemaphoreType.DMA,
            pltpu.SemaphoreType.DMA,
            pltpu.SemaphoreType.DMA,
            pltpu.SemaphoreType.DMA,
        ],
    )(tau_arr, hr)
    return out.reshape(N, C, H, W)
